# Initial kernel scaffold; baseline (speedup 1.0000x reference)
import math

import jax
import jax.numpy as jnp
from jax.experimental import pallas as pl
from jax.experimental.pallas import tpu as pltpu
from jax.experimental.pallas import tpu_sc as plsc

VOCAB = 1000000
D_MODEL = 32
LANES = 16
SCALE = math.sqrt(D_MODEL)
WINDOW = 128


def _sc_gather_scale(x_flat, lut):
    n = x_flat.shape[0]
    idx2d = x_flat.reshape(1, n)
    mesh = plsc.VectorSubcoreMesh(core_axis_name="c", subcore_axis_name="s")

    @pl.kernel(
        out_type=jax.ShapeDtypeStruct((n, D_MODEL), jnp.float32),
        mesh=mesh,
    )
    def kernel_fn(lut_hbm, idx_hbm, out_hbm):
        def body(i_vmem, o_vmem):
            pltpu.sync_copy(lut_hbm.at[i_vmem.at[0]], o_vmem)

            @pl.loop(0, WINDOW)
            def _(r):
                for c in range(0, D_MODEL, LANES):
                    slc = (pl.ds(r, 1), pl.ds(c, LANES))
                    o_vmem.at[*slc][...] = o_vmem.at[*slc][...] * SCALE

        pltpu.emit_pipeline(
            body,
            grid=(n // WINDOW,),
            in_specs=[
                pl.BlockSpec((1, WINDOW), index_map=lambda i: (0, i)),
            ],
            out_specs=[
                pl.BlockSpec((WINDOW, D_MODEL), index_map=lambda i: (i, 0)),
            ],
            core_axis_name=("c", "s"),
            dimension_semantics=(pltpu.PARALLEL,),
        )(idx_hbm, out_hbm)

    return kernel_fn(lut, idx2d)


@jax.jit
def kernel(x, lut):
    batch, hist = x.shape
    x_flat = x.reshape(batch * hist).astype(jnp.int32)
    out = _sc_gather_scale(x_flat, lut)
    return out.reshape(batch, hist, D_MODEL)


# speedup vs baseline: 1.0684x; 1.0684x over previous
import math

import jax
import jax.numpy as jnp
from jax.experimental import pallas as pl
from jax.experimental.pallas import tpu as pltpu
from jax.experimental.pallas import tpu_sc as plsc

VOCAB = 1000000
D_MODEL = 32
LANES = 16
SCALE = math.sqrt(D_MODEL)
WINDOW = 128


def _sc_gather_scale(x_flat, lut):
    n = x_flat.shape[0]
    idx2d = x_flat.reshape(1, n)
    mesh = plsc.VectorSubcoreMesh(core_axis_name="c", subcore_axis_name="s")

    @pl.kernel(
        out_type=jax.ShapeDtypeStruct((n, D_MODEL), jnp.float32),
        mesh=mesh,
        compiler_params=pltpu.CompilerParams(use_tc_tiling_on_sc=False),
    )
    def kernel_fn(lut_hbm, idx_hbm, out_hbm):
        def body(i_vmem, o_vmem):
            pltpu.sync_copy(lut_hbm.at[i_vmem.at[0]], o_vmem)

            @pl.loop(0, WINDOW)
            def _(r):
                for c in range(0, D_MODEL, LANES):
                    slc = (pl.ds(r, 1), pl.ds(c, LANES))
                    o_vmem.at[*slc][...] = o_vmem.at[*slc][...] * SCALE

        pltpu.emit_pipeline(
            body,
            grid=(n // WINDOW,),
            in_specs=[
                pl.BlockSpec((1, WINDOW), index_map=lambda i: (0, i)),
            ],
            out_specs=[
                pl.BlockSpec((WINDOW, D_MODEL), index_map=lambda i: (i, 0)),
            ],
            core_axis_name=("c", "s"),
            dimension_semantics=(pltpu.PARALLEL,),
        )(idx_hbm, out_hbm)

    return kernel_fn(lut, idx2d)


@jax.jit
def kernel(x, lut):
    batch, hist = x.shape
    x_flat = x.reshape(batch * hist).astype(jnp.int32)
    out = _sc_gather_scale(x_flat, lut)
    return out.reshape(batch, hist, D_MODEL)


# speedup vs baseline: 1.2167x; 1.1388x over previous
import math

import jax
import jax.numpy as jnp
from jax.experimental import pallas as pl
from jax.experimental.pallas import tpu as pltpu
from jax.experimental.pallas import tpu_sc as plsc

D_MODEL = 32
LANES = 16
SCALE = math.sqrt(D_MODEL)
B_BLK = 4
SPLITS = ((0, 128), (128, 72))


def _sc_gather_scale(x, lut):
    batch, hist = x.shape
    mesh = plsc.VectorSubcoreMesh(core_axis_name="c", subcore_axis_name="s")

    @pl.kernel(
        out_type=jax.ShapeDtypeStruct((batch, hist, D_MODEL), jnp.float32),
        mesh=mesh,
        scratch_types=[pltpu.SemaphoreType.DMA],
        compiler_params=pltpu.CompilerParams(use_tc_tiling_on_sc=False),
    )
    def kernel_fn(lut_hbm, idx_hbm, out_hbm, sem):
        def body(i_vmem, o_vmem):
            copies = []
            for b in range(B_BLK):
                for off, ln in SPLITS:
                    copies.append(
                        pltpu.async_copy(
                            lut_hbm.at[i_vmem.at[b, pl.ds(off, ln)]],
                            o_vmem.at[b, pl.ds(off, ln)],
                            sem,
                        )
                    )
            for c_ in copies:
                c_.wait()
            for b in range(B_BLK):

                @pl.loop(0, hist, step=8)
                def _(r):
                    for dr in range(8):
                        for c in range(0, D_MODEL, LANES):
                            slc = (b, r + dr, pl.ds(c, LANES))
                            o_vmem.at[*slc][...] = o_vmem.at[*slc][...] * SCALE

        pltpu.emit_pipeline(
            body,
            grid=(batch // B_BLK,),
            in_specs=[
                pl.BlockSpec((B_BLK, hist), index_map=lambda i: (i, 0)),
            ],
            out_specs=[
                pl.BlockSpec(
                    (B_BLK, hist, D_MODEL), index_map=lambda i: (i, 0, 0)
                ),
            ],
            core_axis_name=("c", "s"),
            dimension_semantics=(pltpu.PARALLEL,),
        )(idx_hbm, out_hbm)

    return kernel_fn(lut, x)


@jax.jit
def kernel(x, lut):
    return _sc_gather_scale(x.astype(jnp.int32), lut)
